# Initial kernel scaffold; baseline (speedup 1.0000x reference)
import jax
import jax.numpy as jnp
from jax.experimental import pallas as pl


def kernel(atomic_numbers, positions, dst_idx, src_idx, embed, mpW1, mpW2, dW1, db1, dW2, db2, WtS, WtV, Wm, element_bias):
    raise NotImplementedError("write your pallas kernel here")


# speedup vs baseline: 14.0790x; 14.0790x over previous
import dataclasses
import functools
import math

import jax
import jax.numpy as jnp
from jax import lax
from jax.experimental import pallas as pl
from jax.experimental.pallas import tpu as pltpu
from jax.experimental.pallas import tpu_sc as plsc

N = 10000
E = 320000
F = 32
NB = 8
NDCM = 4
L2 = 9
CUT = 5.0
ZMAX = 18

HF = 16
NC = 2
NS = 16
NW = NC * NS
GB = 16

_BINOM = [float(math.comb(NB - 1, k)) for k in range(NB)]
_S3 = math.sqrt(3.0)

f32 = jnp.float32
i32 = jnp.int32


def _sc_params():
    cp = pltpu.CompilerParams()
    fields = pltpu.CompilerParams.__dataclass_fields__
    if "needs_layout_passes" in fields:
        cp = dataclasses.replace(cp, needs_layout_passes=False)
    if "use_tc_tiling_on_sc" in fields:
        cp = dataclasses.replace(cp, use_tc_tiling_on_sc=False)
    return cp


def _rsqrt16(x):
    i = plsc.bitcast(x, i32)
    i = jnp.int32(0x5F3759DF) - lax.shift_right_logical(i, 1)
    y = plsc.bitcast(i, f32)
    for _ in range(4):
        y = y * (jnp.float32(1.5) - jnp.float32(0.5) * x * y * y)
    return y


_EG = E // NW
_STG = 400
_NCHUNK = _EG // _STG
_NGRP = _STG // GB


@functools.cache
def _build_geom_sc():
    mesh = plsc.VectorSubcoreMesh(core_axis_name="c", subcore_axis_name="s")
    return functools.partial(
        pl.kernel,
        out_type=tuple(jax.ShapeDtypeStruct((E,), f32) for _ in range(16)),
        mesh=mesh,
        compiler_params=_sc_params(),
        scratch_types=[
            pltpu.VMEM((N,), f32),
            pltpu.VMEM((N,), f32),
            pltpu.VMEM((N,), f32),
            pltpu.VMEM((_EG,), i32),
            pltpu.VMEM((_EG,), i32),
        ] + [pltpu.VMEM((_STG,), f32) for _ in range(16)],
    )(_geom_body)


def _geom_sc(posx, posy, posz, src, dst):
    return _build_geom_sc()(posx, posy, posz, src, dst)


def _geom_body(px_hbm, py_hbm, pz_hbm, src_hbm, dst_hbm, *rest):
    outs, scr = rest[:16], rest[16:]
    px, py, pz, src_v, dst_v = scr[:5]
    st = scr[5:]
    w = lax.axis_index("c") * NS + lax.axis_index("s")
    base = w * _EG
    pltpu.sync_copy(px_hbm, px)
    pltpu.sync_copy(py_hbm, py)
    pltpu.sync_copy(pz_hbm, pz)
    pltpu.sync_copy(src_hbm.at[pl.ds(base, _EG)], src_v)
    pltpu.sync_copy(dst_hbm.at[pl.ds(base, _EG)], dst_v)

    @pl.loop(0, _NCHUNK)
    def _chunk(c):
        @pl.loop(0, _NGRP)
        def _grp(g):
            j = c * _STG + g * GB
            s16 = src_v[pl.ds(j, GB)]
            d16 = dst_v[pl.ds(j, GB)]
            dx = plsc.load_gather(px, [s16]) - plsc.load_gather(px, [d16])
            dy = plsc.load_gather(py, [s16]) - plsc.load_gather(py, [d16])
            dz = plsc.load_gather(pz, [s16]) - plsc.load_gather(pz, [d16])
            r2 = dx * dx + dy * dy + dz * dz + jnp.float32(1e-12)
            rinv = _rsqrt16(r2)
            r = r2 * rinv
            ux = dx * rinv
            uy = dy * rinv
            uz = dz * rinv
            col = pl.ds(g * GB, GB)
            st[0][col] = ux
            st[1][col] = uy
            st[2][col] = uz
            st[3][col] = jnp.float32(_S3) * ux * uy
            st[4][col] = jnp.float32(_S3) * uy * uz
            st[5][col] = jnp.float32(0.5) * (jnp.float32(3.0) * uz * uz
                                             - jnp.float32(1.0))
            st[6][col] = jnp.float32(_S3) * ux * uz
            st[7][col] = jnp.float32(0.5 * _S3) * (ux * ux - uy * uy)
            t = jnp.float32(1.0) / (jnp.float32(1.0) + r)
            om = jnp.float32(1.0) - t
            rc = r * jnp.float32(1.0 / CUT)
            inside = rc < jnp.float32(1.0)
            den = jnp.float32(1.0) - rc * rc
            den = jnp.where(inside, den, jnp.float32(1.0))
            cut = jnp.exp(jnp.float32(1.0) - jnp.float32(1.0) / den)
            cut = jnp.where(inside, cut, jnp.float32(0.0))
            omp = [None] * NB
            acc = om
            for k in range(NB - 2, -1, -1):
                omp[k] = acc
                acc = acc * om
            tp = cut
            for k in range(NB):
                val = tp * jnp.float32(_BINOM[k])
                if k < NB - 1:
                    val = val * omp[k]
                    tp = tp * t
                st[8 + k][col] = val

        dstc = pl.ds(base + c * _STG, _STG)
        for m in range(16):
            pltpu.sync_copy(st[m], outs[m].at[dstc])


_B = 80
_EPC = E // NS
_NBATCH = _EPC // _B
_RP = N // NS


def _msg_scratch(full):
    ch = L2 if full else 1
    return [
        pltpu.VMEM((_B,), i32),
        pltpu.VMEM((_B,), i32),
        pltpu.VMEM((_B,), i32),
        pltpu.VMEM((_B, HF), f32),
        pltpu.VMEM((_B, HF), f32),
    ] + [pltpu.VMEM((_B,), f32) for _ in range(NB)] + [
        pltpu.VMEM((_B, ch, HF), f32),
        pltpu.VMEM((_B, L2, HF), f32),
        pltpu.VMEM_SHARED((N, L2, HF), f32),
    ]


def _msg_common(x_hbm, g1_hbm, g2_hbm, sphs, src_hbm, dst_hbm, y_hbm,
                src_t, dst_t, srca_t, g1_t, g2_t, sph_t, xs_t, msg_t, acc):
    h = lax.axis_index("c")
    s = lax.axis_index("s")
    pltpu.sync_copy(x_hbm.at[pl.ds(h * N + s * _RP, _RP)],
                    acc.at[pl.ds(s * _RP, _RP)])
    plsc.subcore_barrier()

    ebase = s * _EPC

    @pl.loop(0, _NBATCH)
    def _batch(bi):
        b0 = ebase + bi * _B
        pltpu.sync_copy(src_hbm.at[pl.ds(b0, _B)], src_t)
        pltpu.sync_copy(dst_hbm.at[pl.ds(b0, _B)], dst_t)
        pltpu.sync_copy(g1_hbm.at[pl.ds(h * E + b0, _B)], g1_t)
        pltpu.sync_copy(g2_hbm.at[pl.ds(h * E + b0, _B)], g2_t)
        for m in range(NB):
            pltpu.sync_copy(sphs[m].at[pl.ds(b0, _B)], sph_t[m])
        off = h * jnp.int32(N)
        for k in range(_B // GB):
            sl = pl.ds(k * GB, GB)
            srca_t[sl] = src_t[sl] + off
        pltpu.sync_copy(x_hbm.at[srca_t], xs_t)

        @pl.loop(0, _B // GB)
        def _grp(g):
            svs = [sph_t[m - 1][pl.ds(g * GB, GB)] for m in range(1, L2)]
            for l in range(GB):
                b = g * GB + l
                x0 = xs_t[b, 0, :]
                g1v = g1_t[b, :]
                g2v = g2_t[b, :]
                t = g1v * x0
                msg_t[b, 0, :] = x0 * g2v + t
                for m in range(1, L2):
                    msg_t[b, m, :] = xs_t[b, m, :] * g2v + svs[m - 1][l] * t

        pltpu.sync_copy(msg_t, acc.at[dst_t], add=True)

    plsc.subcore_barrier()
    pltpu.sync_copy(acc.at[pl.ds(s * _RP, _RP)],
                    y_hbm.at[pl.ds(h * N + s * _RP, _RP)])


@functools.cache
def _build_msg_sc():
    mesh = plsc.VectorSubcoreMesh(core_axis_name="c", subcore_axis_name="s")
    return functools.partial(
        pl.kernel,
        out_type=jax.ShapeDtypeStruct((NC * N, L2, HF), f32),
        mesh=mesh,
        compiler_params=_sc_params(),
        scratch_types=_msg_scratch(full=True),
    )(_msg_body)


def _msg_sc(x_flat, g1f, g2f, sph_list, src, dst):
    return _build_msg_sc()(x_flat, g1f, g2f, *sph_list, src, dst)


def _msg_body(x_hbm, g1_hbm, g2_hbm, *rest):
    sphs = rest[:NB]
    src_hbm, dst_hbm, y_hbm = rest[NB:NB + 3]
    scr = rest[NB + 3:]
    src_t, dst_t, srca_t, g1_t, g2_t = scr[:5]
    sph_t = scr[5:5 + NB]
    xs_t, msg_t, acc = scr[5 + NB:]
    _msg_common(x_hbm, g1_hbm, g2_hbm, sphs, src_hbm, dst_hbm, y_hbm,
                src_t, dst_t, srca_t, g1_t, g2_t, sph_t, xs_t, msg_t, acc)


@functools.cache
def _build_msg0_sc():
    mesh = plsc.VectorSubcoreMesh(core_axis_name="c", subcore_axis_name="s")
    return functools.partial(
        pl.kernel,
        out_type=jax.ShapeDtypeStruct((NC * N, L2, HF), f32),
        mesh=mesh,
        compiler_params=_sc_params(),
        scratch_types=_msg_scratch(full=False),
    )(_msg0_body)


def _msg0_sc(x0_flat, xfull_flat, g1f, g2f, sph_list, src, dst):
    return _build_msg0_sc()(x0_flat, xfull_flat, g1f, g2f, *sph_list, src, dst)


def _msg0_body(x0_hbm, xfull_hbm, g1_hbm, g2_hbm, *rest):
    sphs = rest[:NB]
    src_hbm, dst_hbm, y_hbm = rest[NB:NB + 3]
    scr = rest[NB + 3:]
    src_t, dst_t, srca_t, g1_t, g2_t = scr[:5]
    sph_t = scr[5:5 + NB]
    xs_t, msg_t, acc = scr[5 + NB:]
    h = lax.axis_index("c")
    s = lax.axis_index("s")
    pltpu.sync_copy(xfull_hbm.at[pl.ds(h * N + s * _RP, _RP)],
                    acc.at[pl.ds(s * _RP, _RP)])
    plsc.subcore_barrier()

    ebase = s * _EPC

    @pl.loop(0, _NBATCH)
    def _batch(bi):
        b0 = ebase + bi * _B
        pltpu.sync_copy(src_hbm.at[pl.ds(b0, _B)], src_t)
        pltpu.sync_copy(dst_hbm.at[pl.ds(b0, _B)], dst_t)
        pltpu.sync_copy(g1_hbm.at[pl.ds(h * E + b0, _B)], g1_t)
        pltpu.sync_copy(g2_hbm.at[pl.ds(h * E + b0, _B)], g2_t)
        for m in range(NB):
            pltpu.sync_copy(sphs[m].at[pl.ds(b0, _B)], sph_t[m])
        off = h * jnp.int32(N)
        for k in range(_B // GB):
            sl = pl.ds(k * GB, GB)
            srca_t[sl] = src_t[sl] + off
        pltpu.sync_copy(x0_hbm.at[srca_t], xs_t)

        @pl.loop(0, _B // GB)
        def _grp(g):
            svs = [sph_t[m - 1][pl.ds(g * GB, GB)] for m in range(1, L2)]
            for l in range(GB):
                b = g * GB + l
                x0 = xs_t[b, 0, :]
                t = g1_t[b, :] * x0
                msg_t[b, 0, :] = x0 * g2_t[b, :] + t
                for m in range(1, L2):
                    msg_t[b, m, :] = svs[m - 1][l] * t

        pltpu.sync_copy(msg_t, acc.at[dst_t], add=True)

    plsc.subcore_barrier()
    pltpu.sync_copy(acc.at[pl.ds(s * _RP, _RP)],
                    y_hbm.at[pl.ds(h * N + s * _RP, _RP)])


_EBLK = 1280
_NBLK = 200


def _g_body(*refs):
    rb_refs = refs[:NB]
    w1_ref, w2_ref = refs[NB:NB + 2]
    out_refs = refs[NB + 2:]
    rbb = jnp.concatenate([r[...] for r in rb_refs], axis=0)
    dn = (((0,), (0,)), ((), ()))
    for l in range(3):
        g1_ref = out_refs[l]
        g2_ref = out_refs[3 + l]
        for h in range(NC):
            cols = slice(h * HF, (h + 1) * HF)
            g1_ref[h] = lax.dot_general(rbb, w1_ref[l][:, cols], dn,
                                        preferred_element_type=f32)
            g2_ref[h] = lax.dot_general(rbb, w2_ref[l][:, cols], dn,
                                        preferred_element_type=f32)


def _gates_tc(rb_list, mpW1, mpW2):
    grid = E // _EBLK
    gspec = pl.BlockSpec((NC, _EBLK, HF), lambda i: (0, i, 0))
    rb2d = [r.reshape(1, E) for r in rb_list]
    outs = pl.pallas_call(
        _g_body,
        grid=(grid,),
        in_specs=[pl.BlockSpec((1, _EBLK), lambda i: (0, i))] * NB + [
            pl.BlockSpec((3, NB, F), lambda i: (0, 0, 0)),
            pl.BlockSpec((3, NB, F), lambda i: (0, 0, 0)),
        ],
        out_specs=[gspec] * 6,
        out_shape=[jax.ShapeDtypeStruct((NC, E, HF), f32)] * 6,
    )(*rb2d, mpW1, mpW2)
    return outs[:3], outs[3:]


def _embed_body(z_ref, emb_ref, eb_ref, x2_ref, x0c_ref, ebias_ref):
    z = z_ref[...]
    oh = (lax.broadcasted_iota(i32, (_NBLK, 128), 1) == z).astype(f32)
    x0 = jnp.dot(oh, emb_ref[...], preferred_element_type=f32)
    eb = jnp.sum(oh * eb_ref[...], axis=1, keepdims=True)
    x2_ref[...] = jnp.zeros((NC, _NBLK, L2, HF), f32)
    x2_ref[0, :, 0, :] = x0[:, :HF]
    x2_ref[1, :, 0, :] = x0[:, HF:]
    x0c_ref[0] = x0[:, :HF]
    x0c_ref[1] = x0[:, HF:]
    ebias_ref[...] = eb


def _embed_tc(z2d, embed_pad, ebias_row):
    grid = N // _NBLK
    return pl.pallas_call(
        _embed_body,
        grid=(grid,),
        in_specs=[
            pl.BlockSpec((_NBLK, 1), lambda i: (i, 0)),
            pl.BlockSpec((128, F), lambda i: (0, 0)),
            pl.BlockSpec((1, 128), lambda i: (0, 0)),
        ],
        out_specs=[
            pl.BlockSpec((NC, _NBLK, L2, HF), lambda i: (0, i, 0, 0)),
            pl.BlockSpec((NC, _NBLK, HF), lambda i: (0, i, 0)),
            pl.BlockSpec((_NBLK, 1), lambda i: (i, 0)),
        ],
        out_shape=[
            jax.ShapeDtypeStruct((NC, N, L2, HF), f32),
            jax.ShapeDtypeStruct((NC, N, HF), f32),
            jax.ShapeDtypeStruct((N, 1), f32),
        ],
    )(z2d, embed_pad, ebias_row)


def _deg(y, w_ref, b):
    dn = (((2,), (0,)), ((), ()))
    o0 = jnp.dot(y[:, 0, :], w_ref[0], preferred_element_type=f32) + b
    o1 = lax.dot_general(y[:, 1:4, :], w_ref[1], dn, preferred_element_type=f32)
    o2 = lax.dot_general(y[:, 4:9, :], w_ref[2], dn, preferred_element_type=f32)
    return jnp.concatenate([o0[:, None, :], o1, o2], axis=1)


def _silu(h):
    s = h[:, 0:1, :]
    sg = jax.nn.sigmoid(s)
    gate = sg * (1.0 + s * (1.0 - sg))
    return jnp.concatenate([s * sg, h[:, 1:, :] * gate], axis=1)


def _dense_body(y2_ref, x2_ref, w1_ref, b1_ref, w2_ref, b2_ref, o_ref):
    y = jnp.concatenate([y2_ref[0], y2_ref[1]], axis=-1)
    x = jnp.concatenate([x2_ref[0], x2_ref[1]], axis=-1)
    hh = _deg(y, w1_ref, b1_ref[0])
    hh = _silu(hh)
    hh = _deg(hh, w2_ref, b2_ref[0])
    xn = x + hh
    o_ref[0] = xn[..., :HF]
    o_ref[1] = xn[..., HF:]


def _dense_tc(y2, x2, w1, b1, w2, b2):
    grid = N // _NBLK
    spec = pl.BlockSpec((NC, _NBLK, L2, HF), lambda i: (0, i, 0, 0))
    return pl.pallas_call(
        _dense_body,
        grid=(grid,),
        in_specs=[
            spec,
            spec,
            pl.BlockSpec((3, F, F), lambda i: (0, 0, 0)),
            pl.BlockSpec((1, F), lambda i: (0, 0)),
            pl.BlockSpec((3, F, F), lambda i: (0, 0, 0)),
            pl.BlockSpec((1, F), lambda i: (0, 0)),
        ],
        out_specs=spec,
        out_shape=jax.ShapeDtypeStruct((NC, N, L2, HF), f32),
    )(y2, x2, w1, b1, w2, b2)


def _out_body(x2_ref, eb_ref, wms_ref, wtv_ref, mono_ref, dipo_ref):
    x = jnp.concatenate([x2_ref[0], x2_ref[1]], axis=-1)
    x0 = x[:, 0, :]
    mono_ref[...] = (jnp.dot(x0, wms_ref[...], preferred_element_type=f32)
                     + eb_ref[...])
    dn = (((2,), (0,)), ((), ()))
    vec = lax.dot_general(x[:, 1:4, :], wtv_ref[...], dn,
                          preferred_element_type=f32)
    dipo_ref[...] = jnp.clip(vec, -1.0, 1.0) * jnp.float32(0.3)


def _out_tc(x2, ebias, WmS, WtV):
    grid = N // _NBLK
    return pl.pallas_call(
        _out_body,
        grid=(grid,),
        in_specs=[
            pl.BlockSpec((NC, _NBLK, L2, HF), lambda i: (0, i, 0, 0)),
            pl.BlockSpec((_NBLK, 1), lambda i: (i, 0)),
            pl.BlockSpec((F, NDCM), lambda i: (0, 0)),
            pl.BlockSpec((F, NDCM), lambda i: (0, 0)),
        ],
        out_specs=[
            pl.BlockSpec((_NBLK, NDCM), lambda i: (i, 0)),
            pl.BlockSpec((_NBLK, 3, NDCM), lambda i: (i, 0, 0)),
        ],
        out_shape=[
            jax.ShapeDtypeStruct((N, NDCM), f32),
            jax.ShapeDtypeStruct((N, 3, NDCM), f32),
        ],
    )(x2, ebias, WmS, WtV)


def kernel(atomic_numbers, positions, dst_idx, src_idx, embed, mpW1, mpW2,
           dW1, db1, dW2, db2, WtS, WtV, Wm, element_bias):
    src = src_idx.astype(i32)
    dst = dst_idx.astype(i32)
    posx = positions[:, 0].astype(f32)
    posy = positions[:, 1].astype(f32)
    posz = positions[:, 2].astype(f32)
    z2d = atomic_numbers.astype(i32).reshape(N, 1)
    embed_pad = jnp.zeros((128, F), f32).at[:ZMAX].set(embed)
    ebias_row = jnp.zeros((1, 128), f32).at[0, :ZMAX].set(element_bias)
    WmS = WtS @ Wm

    geom = _geom_sc(posx, posy, posz, src, dst)
    sph_list, rb_list = geom[:8], geom[8:]
    g1s, g2s = _gates_tc(rb_list, mpW1, mpW2)
    x2, x0c, ebias = _embed_tc(z2d, embed_pad, ebias_row)

    y_flat = _msg0_sc(x0c.reshape(NC * N, 1, HF),
                      x2.reshape(NC * N, L2, HF),
                      g1s[0].reshape(NC * E, HF), g2s[0].reshape(NC * E, HF),
                      sph_list, src, dst)
    x2 = _dense_tc(y_flat.reshape(NC, N, L2, HF), x2,
                   dW1[0], db1[0].reshape(1, F), dW2[0], db2[0].reshape(1, F))
    for l in (1, 2):
        y_flat = _msg_sc(x2.reshape(NC * N, L2, HF),
                         g1s[l].reshape(NC * E, HF), g2s[l].reshape(NC * E, HF),
                         sph_list, src, dst)
        x2 = _dense_tc(y_flat.reshape(NC, N, L2, HF), x2,
                       dW1[l], db1[l].reshape(1, F),
                       dW2[l], db2[l].reshape(1, F))

    mono, dipo = _out_tc(x2, ebias, WmS, WtV)
    return (mono, dipo)


# speedup vs baseline: 22.0020x; 1.5628x over previous
import dataclasses
import functools
import math

import jax
import jax.numpy as jnp
from jax import lax
from jax.experimental import pallas as pl
from jax.experimental.pallas import tpu as pltpu
from jax.experimental.pallas import tpu_sc as plsc

N = 10000
E = 320000
F = 32
NB = 8
NDCM = 4
L2 = 9
CUT = 5.0
ZMAX = 18

HF = 16
NC = 2
NS = 16
NW = NC * NS
GB = 16

_BINOM = [float(math.comb(NB - 1, k)) for k in range(NB)]
_S3 = math.sqrt(3.0)

f32 = jnp.float32
i32 = jnp.int32


def _sc_params():
    cp = pltpu.CompilerParams()
    fields = pltpu.CompilerParams.__dataclass_fields__
    if "needs_layout_passes" in fields:
        cp = dataclasses.replace(cp, needs_layout_passes=False)
    if "use_tc_tiling_on_sc" in fields:
        cp = dataclasses.replace(cp, use_tc_tiling_on_sc=False)
    return cp


def _rsqrt16(x):
    i = plsc.bitcast(x, i32)
    i = jnp.int32(0x5F3759DF) - lax.shift_right_logical(i, 1)
    y = plsc.bitcast(i, f32)
    for _ in range(4):
        y = y * (jnp.float32(1.5) - jnp.float32(0.5) * x * y * y)
    return y


_EG = E // NW
_STG = 400
_NCHUNK = _EG // _STG
_NGRP = _STG // GB


@functools.cache
def _build_geom_sc():
    mesh = plsc.VectorSubcoreMesh(core_axis_name="c", subcore_axis_name="s")
    return functools.partial(
        pl.kernel,
        out_type=tuple(jax.ShapeDtypeStruct((E,), f32) for _ in range(16)),
        mesh=mesh,
        compiler_params=_sc_params(),
        scratch_types=[
            pltpu.VMEM((N,), f32),
            pltpu.VMEM((N,), f32),
            pltpu.VMEM((N,), f32),
            pltpu.VMEM((_EG,), i32),
            pltpu.VMEM((_EG,), i32),
        ] + [pltpu.VMEM((_STG,), f32) for _ in range(16)],
    )(_geom_body)


def _geom_sc(posx, posy, posz, src, dst):
    return _build_geom_sc()(posx, posy, posz, src, dst)


def _geom_body(px_hbm, py_hbm, pz_hbm, src_hbm, dst_hbm, *rest):
    outs, scr = rest[:16], rest[16:]
    px, py, pz, src_v, dst_v = scr[:5]
    st = scr[5:]
    w = lax.axis_index("c") * NS + lax.axis_index("s")
    base = w * _EG
    pltpu.sync_copy(px_hbm, px)
    pltpu.sync_copy(py_hbm, py)
    pltpu.sync_copy(pz_hbm, pz)
    pltpu.sync_copy(src_hbm.at[pl.ds(base, _EG)], src_v)
    pltpu.sync_copy(dst_hbm.at[pl.ds(base, _EG)], dst_v)

    @pl.loop(0, _NCHUNK)
    def _chunk(c):
        @pl.loop(0, _NGRP)
        def _grp(g):
            j = c * _STG + g * GB
            s16 = src_v[pl.ds(j, GB)]
            d16 = dst_v[pl.ds(j, GB)]
            dx = plsc.load_gather(px, [s16]) - plsc.load_gather(px, [d16])
            dy = plsc.load_gather(py, [s16]) - plsc.load_gather(py, [d16])
            dz = plsc.load_gather(pz, [s16]) - plsc.load_gather(pz, [d16])
            r2 = dx * dx + dy * dy + dz * dz + jnp.float32(1e-12)
            rinv = _rsqrt16(r2)
            r = r2 * rinv
            ux = dx * rinv
            uy = dy * rinv
            uz = dz * rinv
            col = pl.ds(g * GB, GB)
            st[0][col] = ux
            st[1][col] = uy
            st[2][col] = uz
            st[3][col] = jnp.float32(_S3) * ux * uy
            st[4][col] = jnp.float32(_S3) * uy * uz
            st[5][col] = jnp.float32(0.5) * (jnp.float32(3.0) * uz * uz
                                             - jnp.float32(1.0))
            st[6][col] = jnp.float32(_S3) * ux * uz
            st[7][col] = jnp.float32(0.5 * _S3) * (ux * ux - uy * uy)
            t = jnp.float32(1.0) / (jnp.float32(1.0) + r)
            om = jnp.float32(1.0) - t
            rc = r * jnp.float32(1.0 / CUT)
            inside = rc < jnp.float32(1.0)
            den = jnp.float32(1.0) - rc * rc
            den = jnp.where(inside, den, jnp.float32(1.0))
            cut = jnp.exp(jnp.float32(1.0) - jnp.float32(1.0) / den)
            cut = jnp.where(inside, cut, jnp.float32(0.0))
            omp = [None] * NB
            acc = om
            for k in range(NB - 2, -1, -1):
                omp[k] = acc
                acc = acc * om
            tp = cut
            for k in range(NB):
                val = tp * jnp.float32(_BINOM[k])
                if k < NB - 1:
                    val = val * omp[k]
                    tp = tp * t
                st[8 + k][col] = val

        dstc = pl.ds(base + c * _STG, _STG)
        for m in range(16):
            pltpu.sync_copy(st[m], outs[m].at[dstc])


_B = 80
_S = 400
_BPS = _S // _B
_EPC = E // NS
_NSUP = _EPC // _S
_RP = N // NS


def _msg_scratch(full):
    ch = L2 if full else 1
    return [
        pltpu.VMEM((_S,), i32),
        pltpu.VMEM((_S,), i32),
        pltpu.VMEM((_S, HF), f32),
        pltpu.VMEM((_S, HF), f32),
    ] + [pltpu.VMEM((_S,), f32) for _ in range(NB)] + [
        pltpu.VMEM((_B,), i32),
        pltpu.VMEM((_B,), i32),
        pltpu.VMEM((_B, ch, HF), f32),
        pltpu.VMEM((_B, L2, HF), f32),
        pltpu.VMEM_SHARED((N, L2, HF), f32),
    ]


def _msg_pipeline(x_hbm, xinit_hbm, g1_hbm, g2_hbm, sphs, src_hbm, dst_hbm,
                  y_hbm, scr, full):
    srcs, dsts, g1_s, g2_s = scr[:4]
    sph_s = scr[4:4 + NB]
    srca_t, dst_t, xs_t, msg, acc = scr[4 + NB:]
    h = lax.axis_index("c")
    s = lax.axis_index("s")
    pltpu.sync_copy(xinit_hbm.at[pl.ds(h * N + s * _RP, _RP)],
                    acc.at[pl.ds(s * _RP, _RP)])
    plsc.subcore_barrier()

    off = h * jnp.int32(N)

    @pl.loop(0, _NSUP)
    def _sup(si):
        sb0 = s * _EPC + si * _S
        pltpu.sync_copy(src_hbm.at[pl.ds(sb0, _S)], srcs)
        pltpu.sync_copy(dst_hbm.at[pl.ds(sb0, _S)], dsts)
        pltpu.sync_copy(g1_hbm.at[pl.ds(h * E + sb0, _S)], g1_s)
        pltpu.sync_copy(g2_hbm.at[pl.ds(h * E + sb0, _S)], g2_s)
        for m in range(NB):
            pltpu.sync_copy(sphs[m].at[pl.ds(sb0, _S)], sph_s[m])

        @pl.loop(0, _BPS)
        def _batch(bi):
            for k in range(_B // GB):
                sl_s = pl.ds(bi * _B + k * GB, GB)
                sl = pl.ds(k * GB, GB)
                srca_t[sl] = srcs[sl_s] + off
                dst_t[sl] = dsts[sl_s]
            pltpu.sync_copy(x_hbm.at[srca_t], xs_t)

            @pl.loop(0, _B // GB)
            def _grp(g):
                eb = bi * _B + g * GB
                svs = [sph_s[m][pl.ds(eb, GB)] for m in range(NB)]
                for l in range(GB):
                    b = eb + l
                    bb = g * GB + l
                    x0 = xs_t[bb, 0, :]
                    g1v = g1_s[b, :]
                    g2v = g2_s[b, :]
                    t = g1v * x0
                    msg[bb, 0, :] = x0 * g2v + t
                    for m in range(1, L2):
                        if full:
                            msg[bb, m, :] = (xs_t[bb, m, :] * g2v
                                             + svs[m - 1][l] * t)
                        else:
                            msg[bb, m, :] = svs[m - 1][l] * t

            pltpu.sync_copy(msg, acc.at[dst_t], add=True)

    plsc.subcore_barrier()
    pltpu.sync_copy(acc.at[pl.ds(s * _RP, _RP)],
                    y_hbm.at[pl.ds(h * N + s * _RP, _RP)])


@functools.cache
def _build_msg_sc():
    mesh = plsc.VectorSubcoreMesh(core_axis_name="c", subcore_axis_name="s")
    return functools.partial(
        pl.kernel,
        out_type=jax.ShapeDtypeStruct((NC * N, L2, HF), f32),
        mesh=mesh,
        compiler_params=_sc_params(),
        scratch_types=_msg_scratch(full=True),
    )(_msg_body)


def _msg_sc(x_flat, g1f, g2f, sph_list, src, dst):
    return _build_msg_sc()(x_flat, g1f, g2f, *sph_list, src, dst)


def _msg_body(x_hbm, g1_hbm, g2_hbm, *rest):
    sphs = rest[:NB]
    src_hbm, dst_hbm, y_hbm = rest[NB:NB + 3]
    scr = rest[NB + 3:]
    _msg_pipeline(x_hbm, x_hbm, g1_hbm, g2_hbm, sphs, src_hbm, dst_hbm,
                  y_hbm, scr, full=True)


@functools.cache
def _build_msg0_sc():
    mesh = plsc.VectorSubcoreMesh(core_axis_name="c", subcore_axis_name="s")
    return functools.partial(
        pl.kernel,
        out_type=jax.ShapeDtypeStruct((NC * N, L2, HF), f32),
        mesh=mesh,
        compiler_params=_sc_params(),
        scratch_types=_msg_scratch(full=False),
    )(_msg0_body)


def _msg0_sc(x0_flat, xfull_flat, g1f, g2f, sph_list, src, dst):
    return _build_msg0_sc()(x0_flat, xfull_flat, g1f, g2f, *sph_list, src, dst)


def _msg0_body(x0_hbm, xfull_hbm, g1_hbm, g2_hbm, *rest):
    sphs = rest[:NB]
    src_hbm, dst_hbm, y_hbm = rest[NB:NB + 3]
    scr = rest[NB + 3:]
    _msg_pipeline(x0_hbm, xfull_hbm, g1_hbm, g2_hbm, sphs, src_hbm, dst_hbm,
                  y_hbm, scr, full=False)


_EBLK = 1280
_NBLK = 200


def _g_body(*refs):
    rb_refs = refs[:NB]
    w1_ref, w2_ref = refs[NB:NB + 2]
    out_refs = refs[NB + 2:]
    rbb = jnp.concatenate([r[...] for r in rb_refs], axis=0)
    dn = (((0,), (0,)), ((), ()))
    for l in range(3):
        g1_ref = out_refs[l]
        g2_ref = out_refs[3 + l]
        for h in range(NC):
            cols = slice(h * HF, (h + 1) * HF)
            g1_ref[h] = lax.dot_general(rbb, w1_ref[l][:, cols], dn,
                                        preferred_element_type=f32)
            g2_ref[h] = lax.dot_general(rbb, w2_ref[l][:, cols], dn,
                                        preferred_element_type=f32)


def _gates_tc(rb_list, mpW1, mpW2):
    grid = E // _EBLK
    gspec = pl.BlockSpec((NC, _EBLK, HF), lambda i: (0, i, 0))
    rb2d = [r.reshape(1, E) for r in rb_list]
    outs = pl.pallas_call(
        _g_body,
        grid=(grid,),
        in_specs=[pl.BlockSpec((1, _EBLK), lambda i: (0, i))] * NB + [
            pl.BlockSpec((3, NB, F), lambda i: (0, 0, 0)),
            pl.BlockSpec((3, NB, F), lambda i: (0, 0, 0)),
        ],
        out_specs=[gspec] * 6,
        out_shape=[jax.ShapeDtypeStruct((NC, E, HF), f32)] * 6,
    )(*rb2d, mpW1, mpW2)
    return outs[:3], outs[3:]


def _embed_body(z_ref, emb_ref, eb_ref, x2_ref, x0c_ref, ebias_ref):
    z = z_ref[...]
    oh = (lax.broadcasted_iota(i32, (_NBLK, 128), 1) == z).astype(f32)
    x0 = jnp.dot(oh, emb_ref[...], preferred_element_type=f32)
    eb = jnp.sum(oh * eb_ref[...], axis=1, keepdims=True)
    x2_ref[...] = jnp.zeros((NC, _NBLK, L2, HF), f32)
    x2_ref[0, :, 0, :] = x0[:, :HF]
    x2_ref[1, :, 0, :] = x0[:, HF:]
    x0c_ref[0] = x0[:, :HF]
    x0c_ref[1] = x0[:, HF:]
    ebias_ref[...] = eb


def _embed_tc(z2d, embed_pad, ebias_row):
    grid = N // _NBLK
    return pl.pallas_call(
        _embed_body,
        grid=(grid,),
        in_specs=[
            pl.BlockSpec((_NBLK, 1), lambda i: (i, 0)),
            pl.BlockSpec((128, F), lambda i: (0, 0)),
            pl.BlockSpec((1, 128), lambda i: (0, 0)),
        ],
        out_specs=[
            pl.BlockSpec((NC, _NBLK, L2, HF), lambda i: (0, i, 0, 0)),
            pl.BlockSpec((NC, _NBLK, HF), lambda i: (0, i, 0)),
            pl.BlockSpec((_NBLK, 1), lambda i: (i, 0)),
        ],
        out_shape=[
            jax.ShapeDtypeStruct((NC, N, L2, HF), f32),
            jax.ShapeDtypeStruct((NC, N, HF), f32),
            jax.ShapeDtypeStruct((N, 1), f32),
        ],
    )(z2d, embed_pad, ebias_row)


def _deg(y, w_ref, b):
    dn = (((2,), (0,)), ((), ()))
    o0 = jnp.dot(y[:, 0, :], w_ref[0], preferred_element_type=f32) + b
    o1 = lax.dot_general(y[:, 1:4, :], w_ref[1], dn, preferred_element_type=f32)
    o2 = lax.dot_general(y[:, 4:9, :], w_ref[2], dn, preferred_element_type=f32)
    return jnp.concatenate([o0[:, None, :], o1, o2], axis=1)


def _silu(h):
    s = h[:, 0:1, :]
    sg = jax.nn.sigmoid(s)
    gate = sg * (1.0 + s * (1.0 - sg))
    return jnp.concatenate([s * sg, h[:, 1:, :] * gate], axis=1)


def _dense_body(y2_ref, x2_ref, w1_ref, b1_ref, w2_ref, b2_ref, o_ref):
    y = jnp.concatenate([y2_ref[0], y2_ref[1]], axis=-1)
    x = jnp.concatenate([x2_ref[0], x2_ref[1]], axis=-1)
    hh = _deg(y, w1_ref, b1_ref[0])
    hh = _silu(hh)
    hh = _deg(hh, w2_ref, b2_ref[0])
    xn = x + hh
    o_ref[0] = xn[..., :HF]
    o_ref[1] = xn[..., HF:]


def _dense_tc(y2, x2, w1, b1, w2, b2):
    grid = N // _NBLK
    spec = pl.BlockSpec((NC, _NBLK, L2, HF), lambda i: (0, i, 0, 0))
    return pl.pallas_call(
        _dense_body,
        grid=(grid,),
        in_specs=[
            spec,
            spec,
            pl.BlockSpec((3, F, F), lambda i: (0, 0, 0)),
            pl.BlockSpec((1, F), lambda i: (0, 0)),
            pl.BlockSpec((3, F, F), lambda i: (0, 0, 0)),
            pl.BlockSpec((1, F), lambda i: (0, 0)),
        ],
        out_specs=spec,
        out_shape=jax.ShapeDtypeStruct((NC, N, L2, HF), f32),
    )(y2, x2, w1, b1, w2, b2)


def _out_body(x2_ref, eb_ref, wms_ref, wtv_ref, mono_ref, dipo_ref):
    x = jnp.concatenate([x2_ref[0], x2_ref[1]], axis=-1)
    x0 = x[:, 0, :]
    mono_ref[...] = (jnp.dot(x0, wms_ref[...], preferred_element_type=f32)
                     + eb_ref[...])
    dn = (((2,), (0,)), ((), ()))
    vec = lax.dot_general(x[:, 1:4, :], wtv_ref[...], dn,
                          preferred_element_type=f32)
    dipo_ref[...] = jnp.clip(vec, -1.0, 1.0) * jnp.float32(0.3)


def _out_tc(x2, ebias, WmS, WtV):
    grid = N // _NBLK
    return pl.pallas_call(
        _out_body,
        grid=(grid,),
        in_specs=[
            pl.BlockSpec((NC, _NBLK, L2, HF), lambda i: (0, i, 0, 0)),
            pl.BlockSpec((_NBLK, 1), lambda i: (i, 0)),
            pl.BlockSpec((F, NDCM), lambda i: (0, 0)),
            pl.BlockSpec((F, NDCM), lambda i: (0, 0)),
        ],
        out_specs=[
            pl.BlockSpec((_NBLK, NDCM), lambda i: (i, 0)),
            pl.BlockSpec((_NBLK, 3, NDCM), lambda i: (i, 0, 0)),
        ],
        out_shape=[
            jax.ShapeDtypeStruct((N, NDCM), f32),
            jax.ShapeDtypeStruct((N, 3, NDCM), f32),
        ],
    )(x2, ebias, WmS, WtV)


def kernel(atomic_numbers, positions, dst_idx, src_idx, embed, mpW1, mpW2,
           dW1, db1, dW2, db2, WtS, WtV, Wm, element_bias):
    src = src_idx.astype(i32)
    dst = dst_idx.astype(i32)
    posx = positions[:, 0].astype(f32)
    posy = positions[:, 1].astype(f32)
    posz = positions[:, 2].astype(f32)
    z2d = atomic_numbers.astype(i32).reshape(N, 1)
    embed_pad = jnp.zeros((128, F), f32).at[:ZMAX].set(embed)
    ebias_row = jnp.zeros((1, 128), f32).at[0, :ZMAX].set(element_bias)
    WmS = WtS @ Wm

    geom = _geom_sc(posx, posy, posz, src, dst)
    sph_list, rb_list = geom[:8], geom[8:]
    g1s, g2s = _gates_tc(rb_list, mpW1, mpW2)
    x2, x0c, ebias = _embed_tc(z2d, embed_pad, ebias_row)

    y_flat = _msg0_sc(x0c.reshape(NC * N, 1, HF),
                      x2.reshape(NC * N, L2, HF),
                      g1s[0].reshape(NC * E, HF), g2s[0].reshape(NC * E, HF),
                      sph_list, src, dst)
    x2 = _dense_tc(y_flat.reshape(NC, N, L2, HF), x2,
                   dW1[0], db1[0].reshape(1, F), dW2[0], db2[0].reshape(1, F))
    for l in (1, 2):
        y_flat = _msg_sc(x2.reshape(NC * N, L2, HF),
                         g1s[l].reshape(NC * E, HF), g2s[l].reshape(NC * E, HF),
                         sph_list, src, dst)
        x2 = _dense_tc(y_flat.reshape(NC, N, L2, HF), x2,
                       dW1[l], db1[l].reshape(1, F),
                       dW2[l], db2[l].reshape(1, F))

    mono, dipo = _out_tc(x2, ebias, WmS, WtV)
    return (mono, dipo)


# speedup vs baseline: 25.4405x; 1.1563x over previous
import dataclasses
import functools
import math

import jax
import jax.numpy as jnp
from jax import lax
from jax.experimental import pallas as pl
from jax.experimental.pallas import tpu as pltpu
from jax.experimental.pallas import tpu_sc as plsc

N = 10000
E = 320000
F = 32
NB = 8
NDCM = 4
L2 = 9
CUT = 5.0
ZMAX = 18

HF = 16
NC = 2
NS = 16
NW = NC * NS
GB = 16

_BINOM = [float(math.comb(NB - 1, k)) for k in range(NB)]
_S3 = math.sqrt(3.0)

f32 = jnp.float32
i32 = jnp.int32


def _sc_params():
    cp = pltpu.CompilerParams()
    fields = pltpu.CompilerParams.__dataclass_fields__
    if "needs_layout_passes" in fields:
        cp = dataclasses.replace(cp, needs_layout_passes=False)
    if "use_tc_tiling_on_sc" in fields:
        cp = dataclasses.replace(cp, use_tc_tiling_on_sc=False)
    return cp


def _rsqrt16(x):
    i = plsc.bitcast(x, i32)
    i = jnp.int32(0x5F3759DF) - lax.shift_right_logical(i, 1)
    y = plsc.bitcast(i, f32)
    for _ in range(4):
        y = y * (jnp.float32(1.5) - jnp.float32(0.5) * x * y * y)
    return y


_EG = E // NW
_STG = 400
_NCHUNK = _EG // _STG
_NGRP = _STG // GB


@functools.cache
def _build_geom_sc():
    mesh = plsc.VectorSubcoreMesh(core_axis_name="c", subcore_axis_name="s")
    return functools.partial(
        pl.kernel,
        out_type=tuple(jax.ShapeDtypeStruct((E,), f32) for _ in range(16)),
        mesh=mesh,
        compiler_params=_sc_params(),
        scratch_types=[
            pltpu.VMEM((N,), f32),
            pltpu.VMEM((N,), f32),
            pltpu.VMEM((N,), f32),
            pltpu.VMEM((_EG,), i32),
            pltpu.VMEM((_EG,), i32),
        ] + [pltpu.VMEM((_STG,), f32) for _ in range(16)],
    )(_geom_body)


def _geom_sc(posx, posy, posz, src, dst):
    return _build_geom_sc()(posx, posy, posz, src, dst)


def _geom_body(px_hbm, py_hbm, pz_hbm, src_hbm, dst_hbm, *rest):
    outs, scr = rest[:16], rest[16:]
    px, py, pz, src_v, dst_v = scr[:5]
    st = scr[5:]
    w = lax.axis_index("c") * NS + lax.axis_index("s")
    base = w * _EG
    pltpu.sync_copy(px_hbm, px)
    pltpu.sync_copy(py_hbm, py)
    pltpu.sync_copy(pz_hbm, pz)
    pltpu.sync_copy(src_hbm.at[pl.ds(base, _EG)], src_v)
    pltpu.sync_copy(dst_hbm.at[pl.ds(base, _EG)], dst_v)

    @pl.loop(0, _NCHUNK)
    def _chunk(c):
        @pl.loop(0, _NGRP)
        def _grp(g):
            j = c * _STG + g * GB
            s16 = src_v[pl.ds(j, GB)]
            d16 = dst_v[pl.ds(j, GB)]
            dx = plsc.load_gather(px, [s16]) - plsc.load_gather(px, [d16])
            dy = plsc.load_gather(py, [s16]) - plsc.load_gather(py, [d16])
            dz = plsc.load_gather(pz, [s16]) - plsc.load_gather(pz, [d16])
            r2 = dx * dx + dy * dy + dz * dz + jnp.float32(1e-12)
            rinv = _rsqrt16(r2)
            r = r2 * rinv
            ux = dx * rinv
            uy = dy * rinv
            uz = dz * rinv
            col = pl.ds(g * GB, GB)
            st[0][col] = ux
            st[1][col] = uy
            st[2][col] = uz
            st[3][col] = jnp.float32(_S3) * ux * uy
            st[4][col] = jnp.float32(_S3) * uy * uz
            st[5][col] = jnp.float32(0.5) * (jnp.float32(3.0) * uz * uz
                                             - jnp.float32(1.0))
            st[6][col] = jnp.float32(_S3) * ux * uz
            st[7][col] = jnp.float32(0.5 * _S3) * (ux * ux - uy * uy)
            t = jnp.float32(1.0) / (jnp.float32(1.0) + r)
            om = jnp.float32(1.0) - t
            rc = r * jnp.float32(1.0 / CUT)
            inside = rc < jnp.float32(1.0)
            den = jnp.float32(1.0) - rc * rc
            den = jnp.where(inside, den, jnp.float32(1.0))
            cut = jnp.exp(jnp.float32(1.0) - jnp.float32(1.0) / den)
            cut = jnp.where(inside, cut, jnp.float32(0.0))
            omp = [None] * NB
            acc = om
            for k in range(NB - 2, -1, -1):
                omp[k] = acc
                acc = acc * om
            tp = cut
            for k in range(NB):
                val = tp * jnp.float32(_BINOM[k])
                if k < NB - 1:
                    val = val * omp[k]
                    tp = tp * t
                st[8 + k][col] = val

        dstc = pl.ds(base + c * _STG, _STG)
        for m in range(16):
            pltpu.sync_copy(st[m], outs[m].at[dstc])


_B = 80
_S = 400
_BPS = _S // _B
_EPC = E // NS
_NSUP = _EPC // _S
_RP = N // NS


def _msg_scratch(full):
    ch = L2 if full else 1
    return [
        pltpu.VMEM((_S,), i32),
        pltpu.VMEM((_S,), i32),
        pltpu.VMEM((_S, 32), f32),
    ] + [pltpu.VMEM((_S,), f32) for _ in range(NB)] + [
        pltpu.VMEM((_B,), i32),
        pltpu.VMEM((_B,), i32),
        pltpu.VMEM((_B, ch, HF), f32),
        pltpu.VMEM((_B, L2, HF), f32),
        pltpu.VMEM_SHARED((N, L2, HF), f32),
    ]


def _msg_pipeline(x_hbm, xinit_hbm, ga_hbm, gb_hbm, sphs, src_hbm, dst_hbm,
                  y_hbm, scr, layer, full):
    srcs, dsts, g_s = scr[:3]
    sph_s = scr[3:3 + NB]
    srca_t, dst_t, xs_t, msg, acc = scr[3 + NB:]
    h = lax.axis_index("c")
    s = lax.axis_index("s")
    pltpu.sync_copy(xinit_hbm.at[pl.ds(h * N + s * _RP, _RP)],
                    acc.at[pl.ds(s * _RP, _RP)])
    plsc.subcore_barrier()

    off = h * jnp.int32(N)
    gc = pl.ds(layer * 32, 32)

    @pl.loop(0, _NSUP)
    def _sup(si):
        sb0 = s * _EPC + si * _S
        pltpu.sync_copy(src_hbm.at[pl.ds(sb0, _S)], srcs)
        pltpu.sync_copy(dst_hbm.at[pl.ds(sb0, _S)], dsts)

        @pl.when(h == 0)
        def _ga():
            pltpu.sync_copy(ga_hbm.at[pl.ds(sb0, _S), gc], g_s)

        @pl.when(h == 1)
        def _gb():
            pltpu.sync_copy(gb_hbm.at[pl.ds(sb0, _S), gc], g_s)

        for m in range(NB):
            pltpu.sync_copy(sphs[m].at[pl.ds(sb0, _S)], sph_s[m])

        @pl.loop(0, _BPS)
        def _batch(bi):
            for k in range(_B // GB):
                sl_s = pl.ds(bi * _B + k * GB, GB)
                sl = pl.ds(k * GB, GB)
                srca_t[sl] = srcs[sl_s] + off
                dst_t[sl] = dsts[sl_s]
            pltpu.sync_copy(x_hbm.at[srca_t], xs_t)

            @pl.loop(0, _B // GB)
            def _grp(g):
                eb = bi * _B + g * GB
                svs = [sph_s[m][pl.ds(eb, GB)] for m in range(NB)]
                for l in range(GB):
                    b = eb + l
                    bb = g * GB + l
                    x0 = xs_t[bb, 0, :]
                    g1v = g_s[b, 0:HF]
                    g2v = g_s[b, HF:2 * HF]
                    t = g1v * x0
                    msg[bb, 0, :] = x0 * g2v + t
                    for m in range(1, L2):
                        if full:
                            msg[bb, m, :] = (xs_t[bb, m, :] * g2v
                                             + svs[m - 1][l] * t)
                        else:
                            msg[bb, m, :] = svs[m - 1][l] * t

            pltpu.sync_copy(msg, acc.at[dst_t], add=True)

    plsc.subcore_barrier()
    pltpu.sync_copy(acc.at[pl.ds(s * _RP, _RP)],
                    y_hbm.at[pl.ds(h * N + s * _RP, _RP)])


@functools.cache
def _build_msg_sc(layer):
    mesh = plsc.VectorSubcoreMesh(core_axis_name="c", subcore_axis_name="s")

    def body(x_hbm, ga_hbm, gb_hbm, *rest):
        sphs = rest[:NB]
        src_hbm, dst_hbm, y_hbm = rest[NB:NB + 3]
        scr = rest[NB + 3:]
        _msg_pipeline(x_hbm, x_hbm, ga_hbm, gb_hbm, sphs, src_hbm, dst_hbm,
                      y_hbm, scr, layer, full=True)

    return functools.partial(
        pl.kernel,
        out_type=jax.ShapeDtypeStruct((NC * N, L2, HF), f32),
        mesh=mesh,
        compiler_params=_sc_params(),
        scratch_types=_msg_scratch(full=True),
    )(body)


def _msg_sc(layer, x_flat, ga, gb, sph_list, src, dst):
    return _build_msg_sc(layer)(x_flat, ga, gb, *sph_list, src, dst)


@functools.cache
def _build_msg0_sc():
    mesh = plsc.VectorSubcoreMesh(core_axis_name="c", subcore_axis_name="s")

    def body(x0_hbm, xfull_hbm, ga_hbm, gb_hbm, *rest):
        sphs = rest[:NB]
        src_hbm, dst_hbm, y_hbm = rest[NB:NB + 3]
        scr = rest[NB + 3:]
        _msg_pipeline(x0_hbm, xfull_hbm, ga_hbm, gb_hbm, sphs, src_hbm,
                      dst_hbm, y_hbm, scr, 0, full=False)

    return functools.partial(
        pl.kernel,
        out_type=jax.ShapeDtypeStruct((NC * N, L2, HF), f32),
        mesh=mesh,
        compiler_params=_sc_params(),
        scratch_types=_msg_scratch(full=False),
    )(body)


def _msg0_sc(x0_flat, xfull_flat, ga, gb, sph_list, src, dst):
    return _build_msg0_sc()(x0_flat, xfull_flat, ga, gb, *sph_list, src, dst)


_EBLK = 1280
_NBLK = 1000
_CH = L2 * HF


def _g_body(*refs):
    rb_refs = refs[:NB]
    w1_ref, w2_ref = refs[NB:NB + 2]
    ga_ref, gb_ref = refs[NB + 2:]
    rbb = jnp.concatenate([r[...] for r in rb_refs], axis=0)
    dn = (((0,), (0,)), ((), ()))
    cols = {0: [], 1: []}
    for l in range(3):
        for h in range(NC):
            sl = slice(h * HF, (h + 1) * HF)
            cols[h].append(lax.dot_general(rbb, w1_ref[l][:, sl], dn,
                                           preferred_element_type=f32))
            cols[h].append(lax.dot_general(rbb, w2_ref[l][:, sl], dn,
                                           preferred_element_type=f32))
    zpad = jnp.zeros((_EBLK, 32), f32)
    ga_ref[...] = jnp.concatenate(cols[0] + [zpad], axis=1).reshape(_EBLK * 128)
    gb_ref[...] = jnp.concatenate(cols[1] + [zpad], axis=1).reshape(_EBLK * 128)


def _gates_tc(rb_list, mpW1, mpW2):
    grid = E // _EBLK
    rb2d = [r.reshape(1, E) for r in rb_list]
    ga, gb = pl.pallas_call(
        _g_body,
        grid=(grid,),
        in_specs=[pl.BlockSpec((1, _EBLK), lambda i: (0, i))] * NB + [
            pl.BlockSpec((3, NB, F), lambda i: (0, 0, 0)),
            pl.BlockSpec((3, NB, F), lambda i: (0, 0, 0)),
        ],
        out_specs=[pl.BlockSpec((_EBLK * 128,), lambda i: (i,))] * 2,
        out_shape=[jax.ShapeDtypeStruct((E * 128,), f32)] * 2,
    )(*rb2d, mpW1, mpW2)
    return ga.reshape(E, 128), gb.reshape(E, 128)


def _embed_body(z_ref, emb_ref, eb_ref, xa_ref, xb_ref, ebias_ref):
    z = z_ref[...]
    oh = (lax.broadcasted_iota(i32, (_NBLK, 128), 1) == z).astype(f32)
    x0 = jnp.dot(oh, emb_ref[...], preferred_element_type=f32)
    eb = jnp.sum(oh * eb_ref[...], axis=1, keepdims=True)
    zpad = jnp.zeros((_NBLK, _CH - HF), f32)
    xa_ref[...] = jnp.concatenate([x0[:, :HF], zpad], axis=1)
    xb_ref[...] = jnp.concatenate([x0[:, HF:], zpad], axis=1)
    ebias_ref[...] = eb


def _embed_tc(z2d, embed_pad, ebias_row):
    grid = N // _NBLK
    xa, xb, ebias = pl.pallas_call(
        _embed_body,
        grid=(grid,),
        in_specs=[
            pl.BlockSpec((_NBLK, 1), lambda i: (i, 0)),
            pl.BlockSpec((128, F), lambda i: (0, 0)),
            pl.BlockSpec((1, 128), lambda i: (0, 0)),
        ],
        out_specs=[
            pl.BlockSpec((_NBLK, _CH), lambda i: (i, 0)),
            pl.BlockSpec((_NBLK, _CH), lambda i: (i, 0)),
            pl.BlockSpec((_NBLK, 1), lambda i: (i, 0)),
        ],
        out_shape=[
            jax.ShapeDtypeStruct((N, _CH), f32),
            jax.ShapeDtypeStruct((N, _CH), f32),
            jax.ShapeDtypeStruct((N, 1), f32),
        ],
    )(z2d, embed_pad, ebias_row)
    return jnp.concatenate([xa, xb], axis=0), ebias


_GRP = [0] + [1] * 3 + [2] * 5


def _bd_quad(W3, r, c):
    rows = slice(r * HF, (r + 1) * HF)
    cols = slice(c * HF, (c + 1) * HF)
    return jax.scipy.linalg.block_diag(
        *[W3[_GRP[m]][rows, cols] for m in range(L2)])


def _bias_row(b, h):
    return jnp.zeros((1, _CH), f32).at[0, :HF].set(b[h * HF:(h + 1) * HF])


def _dense_body(ya_ref, yb_ref, xa_ref, xb_ref, *wrefs):
    (bd1aa, bd1ab, bd1ba, bd1bb, b1a, b1b,
     bd2aa, bd2ab, bd2ba, bd2bb, b2a, b2b, oa_ref, ob_ref) = wrefs

    def dense(aa, ab, ba, bb, ba_row, bb_row, ua, ub):
        ha = (jnp.dot(ua, aa[...], preferred_element_type=f32)
              + jnp.dot(ub, ba[...], preferred_element_type=f32) + ba_row[...])
        hb = (jnp.dot(ua, ab[...], preferred_element_type=f32)
              + jnp.dot(ub, bb[...], preferred_element_type=f32) + bb_row[...])
        return ha, hb

    ya = ya_ref[...]
    yb = yb_ref[...]
    h1a, h1b = dense(bd1aa, bd1ab, bd1ba, bd1bb, b1a, b1b, ya, yb)
    s = jnp.concatenate([h1a[:, :HF], h1b[:, :HF]], axis=1)
    sg = jax.nn.sigmoid(s)
    gate = sg * (1.0 + s * (1.0 - sg))
    act0 = s * sg
    ga = jnp.concatenate([gate[:, :HF]] * (L2 - 1), axis=1)
    gb = jnp.concatenate([gate[:, HF:]] * (L2 - 1), axis=1)
    h1a = jnp.concatenate([act0[:, :HF], h1a[:, HF:] * ga], axis=1)
    h1b = jnp.concatenate([act0[:, HF:], h1b[:, HF:] * gb], axis=1)
    h2a, h2b = dense(bd2aa, bd2ab, bd2ba, bd2bb, b2a, b2b, h1a, h1b)
    oa_ref[...] = xa_ref[...] + h2a
    ob_ref[...] = xb_ref[...] + h2b


def _dense_tc(y2d, x2d, w1, b1, w2, b2):
    grid = N // _NBLK
    nb = N // _NBLK
    half0 = pl.BlockSpec((_NBLK, _CH), lambda i: (i, 0))
    half1 = pl.BlockSpec((_NBLK, _CH), lambda i: (i + nb, 0))
    wspec = pl.BlockSpec((_CH, _CH), lambda i: (0, 0))
    bspec = pl.BlockSpec((1, _CH), lambda i: (0, 0))
    weights = [_bd_quad(w1, 0, 0), _bd_quad(w1, 0, 1), _bd_quad(w1, 1, 0),
               _bd_quad(w1, 1, 1), _bias_row(b1, 0), _bias_row(b1, 1),
               _bd_quad(w2, 0, 0), _bd_quad(w2, 0, 1), _bd_quad(w2, 1, 0),
               _bd_quad(w2, 1, 1), _bias_row(b2, 0), _bias_row(b2, 1)]
    wspecs = [wspec, wspec, wspec, wspec, bspec, bspec] * 2
    oa, ob = pl.pallas_call(
        _dense_body,
        grid=(grid,),
        in_specs=[half0, half1, half0, half1] + wspecs,
        out_specs=[pl.BlockSpec((_NBLK, _CH), lambda i: (i, 0))] * 2,
        out_shape=[jax.ShapeDtypeStruct((N, _CH), f32)] * 2,
    )(y2d, y2d, x2d, x2d, *weights)
    return jnp.concatenate([oa, ob], axis=0)


def _out_body(xa_ref, xb_ref, eb_ref, wms_ref, wtv_ref, mono_ref, dipo_ref):
    xa = xa_ref[...]
    xb = xb_ref[...]
    x0 = jnp.concatenate([xa[:, :HF], xb[:, :HF]], axis=1)
    mono_ref[...] = (jnp.dot(x0, wms_ref[...], preferred_element_type=f32)
                     + eb_ref[...])
    vecs = []
    for c in range(1, 4):
        ch = jnp.concatenate([xa[:, c * HF:(c + 1) * HF],
                              xb[:, c * HF:(c + 1) * HF]], axis=1)
        vecs.append(jnp.dot(ch, wtv_ref[...],
                            preferred_element_type=f32)[:, None, :])
    vec = jnp.concatenate(vecs, axis=1)
    dipo_ref[...] = jnp.clip(vec, -1.0, 1.0) * jnp.float32(0.3)


def _out_tc(x2d, ebias, WmS, WtV):
    grid = N // _NBLK
    nb = N // _NBLK
    return pl.pallas_call(
        _out_body,
        grid=(grid,),
        in_specs=[
            pl.BlockSpec((_NBLK, _CH), lambda i: (i, 0)),
            pl.BlockSpec((_NBLK, _CH), lambda i: (i + nb, 0)),
            pl.BlockSpec((_NBLK, 1), lambda i: (i, 0)),
            pl.BlockSpec((F, NDCM), lambda i: (0, 0)),
            pl.BlockSpec((F, NDCM), lambda i: (0, 0)),
        ],
        out_specs=[
            pl.BlockSpec((_NBLK, NDCM), lambda i: (i, 0)),
            pl.BlockSpec((_NBLK, 3, NDCM), lambda i: (i, 0, 0)),
        ],
        out_shape=[
            jax.ShapeDtypeStruct((N, NDCM), f32),
            jax.ShapeDtypeStruct((N, 3, NDCM), f32),
        ],
    )(x2d, x2d, ebias, WmS, WtV)


def kernel(atomic_numbers, positions, dst_idx, src_idx, embed, mpW1, mpW2,
           dW1, db1, dW2, db2, WtS, WtV, Wm, element_bias):
    src = src_idx.astype(i32)
    dst = dst_idx.astype(i32)
    posx = positions[:, 0].astype(f32)
    posy = positions[:, 1].astype(f32)
    posz = positions[:, 2].astype(f32)
    z2d = atomic_numbers.astype(i32).reshape(N, 1)
    embed_pad = jnp.zeros((128, F), f32).at[:ZMAX].set(embed)
    ebias_row = jnp.zeros((1, 128), f32).at[0, :ZMAX].set(element_bias)
    WmS = WtS @ Wm

    geom = _geom_sc(posx, posy, posz, src, dst)
    sph_list, rb_list = geom[:8], geom[8:]
    ga, gb = _gates_tc(rb_list, mpW1, mpW2)
    x2d, ebias = _embed_tc(z2d, embed_pad, ebias_row)
    x0c = x2d[:, :HF]

    y_flat = _msg0_sc(x0c.reshape(NC * N, 1, HF),
                      x2d.reshape(NC * N, L2, HF),
                      ga, gb, sph_list, src, dst)
    x2d = _dense_tc(y_flat.reshape(NC * N, _CH), x2d,
                    dW1[0], db1[0], dW2[0], db2[0])
    for l in (1, 2):
        y_flat = _msg_sc(l, x2d.reshape(NC * N, L2, HF),
                         ga, gb, sph_list, src, dst)
        x2d = _dense_tc(y_flat.reshape(NC * N, _CH), x2d,
                        dW1[l], db1[l], dW2[l], db2[l])

    mono, dipo = _out_tc(x2d, ebias, WmS, WtV)
    return (mono, dipo)
